# Initial kernel scaffold; baseline (speedup 1.0000x reference)
import jax
import jax.numpy as jnp
from jax.experimental import pallas as pl


def kernel(x, edge_index, edge_attr, W_node, b_node, W_edge, b_edge, conv_eps, conv_W1, conv_b1, conv_W2, conv_b2, mlp_W1, mlp_b1, mlp_W2, mlp_b2):
    raise NotImplementedError("write your pallas kernel here")


# speedup vs baseline: 1.1434x; 1.1434x over previous
import functools

import jax
import jax.numpy as jnp
from jax import lax
from jax.experimental import pallas as pl
from jax.experimental.pallas import tpu as pltpu
from jax.experimental.pallas import tpu_sc as plsc

N = 10000
E = 160000
NODE_IN = 256
EDGE_IN = 16
H = 512
L = 3

NC = 2
NS = 16
NW = NC * NS
EB = 128
BLOCKS = 40
EPW = EB * BLOCKS
EPAD = EPW * NW
CH = 4
CW = 128
NACC = 10240
RPT = NACC // NS
ZROWS = 128
NEG = -1.0e30


def _sc_segment_sum(hs, eas, src3, dst3):
    mesh = plsc.VectorSubcoreMesh(core_axis_name="c", subcore_axis_name="s")

    @functools.partial(
        pl.kernel,
        out_type=jax.ShapeDtypeStruct((NC, CH, NACC, CW), jnp.float32),
        mesh=mesh,
        scratch_types=[
            pltpu.VMEM((1, EB), jnp.int32),
            pltpu.VMEM((1, EB), jnp.int32),
            pltpu.VMEM((EB, CW), jnp.float32),
            pltpu.VMEM((EB, CW), jnp.float32),
            pltpu.VMEM_SHARED((NACC, CW), jnp.float32),
            pltpu.SemaphoreType.DMA,
        ],
    )
    def k(h0, h1, h2, h3, e0, e1, e2, e3, src_h, dst_h, out_h,
          src_v, dst_v, gath, eab, accum, sem):
        cid = lax.axis_index("c")
        sid = lax.axis_index("s")
        w = cid * NS + sid

        hts = (h0, h1, h2, h3)
        ets = (e0, e1, e2, e3)
        for c in range(CH):
            @pl.loop(0, ZROWS)
            def _zrow(r):
                for j in range(CW // 16):
                    eab[r, pl.ds(j * 16, 16)] = jnp.zeros((16,), jnp.float32)

            for z in range(RPT // ZROWS):
                pltpu.sync_copy(eab, accum.at[pl.ds(sid * RPT + z * ZROWS, ZROWS)])
            plsc.subcore_barrier()

            @pl.loop(0, BLOCKS)
            def _blk(b):
                base = w * EPW + b * EB
                pltpu.sync_copy(src_h.at[w, b], src_v.at[0])
                pltpu.sync_copy(dst_h.at[w, b], dst_v.at[0])
                pltpu.async_copy(hts[c].at[src_v.at[0]], gath, sem).wait()
                pltpu.sync_copy(ets[c].at[pl.ds(base, EB)], eab)

                @pl.loop(0, EB)
                def _row(r):
                    for j in range(CW // 16):
                        s = pl.ds(j * 16, 16)
                        gath[r, s] = jnp.maximum(gath[r, s] + eab[r, s], 0.0)

                pltpu.sync_copy(gath, accum.at[dst_v.at[0]], add=True)

            plsc.subcore_barrier()
            pltpu.sync_copy(accum.at[pl.ds(sid * RPT, RPT)],
                            out_h.at[cid, c, pl.ds(sid * RPT, RPT)])
            plsc.subcore_barrier()

    return k(*hs, *eas, src3, dst3)


_RB = 1000


def _node_proj(x, W, b):
    def body(x_ref, w_ref, b_ref, o0, o1, o2, o3):
        v = jnp.dot(x_ref[...], w_ref[...],
                    preferred_element_type=jnp.float32) + b_ref[...]
        for c, o in enumerate((o0, o1, o2, o3)):
            o[...] = v[:, c * CW:(c + 1) * CW]

    return pl.pallas_call(
        body,
        grid=(N // _RB,),
        in_specs=[
            pl.BlockSpec((_RB, NODE_IN), lambda i: (i, 0)),
            pl.BlockSpec((NODE_IN, H), lambda i: (0, 0)),
            pl.BlockSpec((1, H), lambda i: (0, 0)),
        ],
        out_specs=[pl.BlockSpec((_RB, CW), lambda i: (i, 0))] * CH,
        out_shape=[jax.ShapeDtypeStruct((N, CW), jnp.float32)] * CH,
    )(x, W, b.reshape(1, H))


def _edge_proj_chunk(eap, Wc, bc):
    BE = 2048

    def body(a_ref, w_ref, b_ref, o_ref):
        e = pl.program_id(0)
        v = jnp.dot(a_ref[...], w_ref[...],
                    preferred_element_type=jnp.float32) + b_ref[...]
        row = e * BE + lax.broadcasted_iota(jnp.int32, (BE, CW), 0)
        o_ref[...] = jnp.where(row < E, v, NEG)

    return pl.pallas_call(
        body,
        grid=(EPAD // BE,),
        in_specs=[
            pl.BlockSpec((BE, EDGE_IN), lambda e: (e, 0)),
            pl.BlockSpec((EDGE_IN, CW), lambda e: (0, 0)),
            pl.BlockSpec((1, CW), lambda e: (0, 0)),
        ],
        out_specs=pl.BlockSpec((BE, CW), lambda e: (e, 0)),
        out_shape=jax.ShapeDtypeStruct((EPAD, CW), jnp.float32),
    )(eap, Wc, bc.reshape(1, CW))


def _layer_dense(hs, part, scale, W1, b1, W2, b2, last):
    def body(s_ref, h0, h1, h2, h3, p_ref, w1_ref, b1_ref, w2_ref, b2_ref,
             o0, o1, o2, o3):
        hh = (h0, h1, h2, h3)
        acc = jnp.zeros((_RB, 2 * H), jnp.float32)
        for c in range(CH):
            z = s_ref[...] * hh[c][...] + p_ref[0, c] + p_ref[1, c]
            acc += jnp.dot(z, w1_ref[c * CW:(c + 1) * CW, :],
                           preferred_element_type=jnp.float32)
        z1 = jnp.maximum(acc + b1_ref[...], 0.0)
        h2v = jnp.dot(z1, w2_ref[...], preferred_element_type=jnp.float32)
        h2v = h2v + b2_ref[...]
        if not last:
            h2v = jnp.maximum(h2v, 0.0)
        for c, o in enumerate((o0, o1, o2, o3)):
            o[...] = h2v[:, c * CW:(c + 1) * CW]

    return pl.pallas_call(
        body,
        grid=(N // _RB,),
        in_specs=[
            pl.BlockSpec((1, CW), lambda i: (0, 0)),
            pl.BlockSpec((_RB, CW), lambda i: (i, 0)),
            pl.BlockSpec((_RB, CW), lambda i: (i, 0)),
            pl.BlockSpec((_RB, CW), lambda i: (i, 0)),
            pl.BlockSpec((_RB, CW), lambda i: (i, 0)),
            pl.BlockSpec((NC, CH, _RB, CW), lambda i: (0, 0, i, 0)),
            pl.BlockSpec((H, 2 * H), lambda i: (0, 0)),
            pl.BlockSpec((1, 2 * H), lambda i: (0, 0)),
            pl.BlockSpec((2 * H, H), lambda i: (0, 0)),
            pl.BlockSpec((1, H), lambda i: (0, 0)),
        ],
        out_specs=[pl.BlockSpec((_RB, CW), lambda i: (i, 0))] * CH,
        out_shape=[jax.ShapeDtypeStruct((N, CW), jnp.float32)] * CH,
    )(scale, *hs, part, W1, b1.reshape(1, 2 * H), W2, b2.reshape(1, H))


def _final_mlp(hs, W1, b1, W2p, b2p):
    def body(h0, h1, h2, h3, w1_ref, b1_ref, w2_ref, b2_ref, o_ref):
        hh = (h0, h1, h2, h3)
        acc = jnp.zeros((_RB, H), jnp.float32)
        for c in range(CH):
            acc += jnp.dot(hh[c][...], w1_ref[c * CW:(c + 1) * CW, :],
                           preferred_element_type=jnp.float32)
        z = jnp.maximum(acc + b1_ref[...], 0.0)
        o = jnp.dot(z, w2_ref[...], preferred_element_type=jnp.float32)
        o_ref[...] = jax.nn.sigmoid(o + b2_ref[...])

    return pl.pallas_call(
        body,
        grid=(N // _RB,),
        in_specs=[
            pl.BlockSpec((_RB, CW), lambda i: (i, 0)),
            pl.BlockSpec((_RB, CW), lambda i: (i, 0)),
            pl.BlockSpec((_RB, CW), lambda i: (i, 0)),
            pl.BlockSpec((_RB, CW), lambda i: (i, 0)),
            pl.BlockSpec((H, H), lambda i: (0, 0)),
            pl.BlockSpec((1, H), lambda i: (0, 0)),
            pl.BlockSpec((H, CW), lambda i: (0, 0)),
            pl.BlockSpec((1, CW), lambda i: (0, 0)),
        ],
        out_specs=pl.BlockSpec((_RB, CW), lambda i: (i, 0)),
        out_shape=jax.ShapeDtypeStruct((N, CW), jnp.float32),
    )(*hs, W1, b1.reshape(1, H), W2p, b2p)


def kernel(x, edge_index, edge_attr, W_node, b_node, W_edge, b_edge,
           conv_eps, conv_W1, conv_b1, conv_W2, conv_b2,
           mlp_W1, mlp_b1, mlp_W2, mlp_b2):
    src = edge_index[0]
    dst = edge_index[1]
    pad = EPAD - E
    src3 = jnp.concatenate([src, jnp.zeros((pad,), jnp.int32)]).reshape(
        NW, BLOCKS, EB)
    dst3 = jnp.concatenate([dst, jnp.zeros((pad,), jnp.int32)]).reshape(
        NW, BLOCKS, EB)
    eap = jnp.concatenate(
        [edge_attr, jnp.zeros((pad, EDGE_IN), jnp.float32)], axis=0)

    hs = _node_proj(x, W_node, b_node)
    eas = tuple(
        _edge_proj_chunk(eap, W_edge[:, c * CW:(c + 1) * CW],
                         b_edge[c * CW:(c + 1) * CW])
        for c in range(CH))

    for l in range(L):
        part = _sc_segment_sum(hs, eas, src3, dst3)
        scale = jnp.full((1, CW), 1.0, jnp.float32) + conv_eps[l]
        hs = _layer_dense(hs, part, scale, conv_W1[l], conv_b1[l],
                          conv_W2[l], conv_b2[l], last=(l == L - 1))

    W2p = jnp.pad(mlp_W2, ((0, 0), (0, CW - 1)))
    b2p = jnp.pad(mlp_b2, (0, CW - 1)).reshape(1, CW)
    o = _final_mlp(hs, mlp_W1, mlp_b1, W2p, b2p)
    return o[:, 0]


# speedup vs baseline: 1.7289x; 1.5121x over previous
import functools

import jax
import jax.numpy as jnp
from jax import lax
from jax.experimental import pallas as pl
from jax.experimental.pallas import tpu as pltpu
from jax.experimental.pallas import tpu_sc as plsc

N = 10000
E = 160000
NODE_IN = 256
EDGE_IN = 16
H = 512
L = 3

NC = 2
NS = 16
NW = NC * NS
EB = 64
BLOCKS = 80
EPW = EB * BLOCKS
EPAD = EPW * NW
CH = 4
CW = 128
NACC = 10240
RPT = NACC // NS
NEG = -1.0e30


def _sc_segment_sum(hs, eas, src3, dst3):
    mesh = plsc.VectorSubcoreMesh(core_axis_name="c", subcore_axis_name="s")

    @functools.partial(
        pl.kernel,
        out_type=jax.ShapeDtypeStruct((NC, CH, NACC, CW), jnp.float32),
        mesh=mesh,
        scratch_types=[
            pltpu.VMEM((BLOCKS, EB), jnp.int32),
            pltpu.VMEM((2, EB), jnp.int32),
            pltpu.VMEM((EB, CW), jnp.float32),
            pltpu.VMEM((EB, CW), jnp.float32),
            pltpu.VMEM((EB, CW), jnp.float32),
            pltpu.VMEM((EB, CW), jnp.float32),
            pltpu.VMEM_SHARED((NACC, CW), jnp.float32),
            pltpu.SemaphoreType.DMA,
            pltpu.SemaphoreType.DMA,
            pltpu.SemaphoreType.DMA,
            pltpu.SemaphoreType.DMA,
            pltpu.SemaphoreType.DMA,
            pltpu.SemaphoreType.DMA,
        ],
    )
    def k(h0, h1, h2, h3, e0, e1, e2, e3, src_h, dst_h, out_h,
          src_v, dst_i, g0, g1, a0, a1, accum,
          sg0, sg1, se0, se1, sd0, sd1):
        cid = lax.axis_index("c")
        sid = lax.axis_index("s")
        w = cid * NS + sid
        ebase = w * EPW

        pltpu.sync_copy(src_h.at[w], src_v)

        hts = (h0, h1, h2, h3)
        ets = (e0, e1, e2, e3)
        slots = ((g0, a0, sg0, se0, sd0), (g1, a1, sg1, se1, sd1))
        for c in range(CH):
            ht = hts[c]
            et = ets[c]

            @pl.loop(0, EB)
            def _zrow(r):
                for j in range(CW // 16):
                    a0[r, pl.ds(j * 16, 16)] = jnp.zeros((16,), jnp.float32)

            for z in range(RPT // EB):
                pltpu.sync_copy(a0, accum.at[pl.ds(sid * RPT + z * EB, EB)])
            plsc.subcore_barrier()

            for s, (gb, ab, sg, se, sd) in enumerate(slots):
                pltpu.async_copy(ht.at[src_v.at[s]], gb, sg)
                pltpu.async_copy(et.at[pl.ds(ebase + s * EB, EB)], ab, se)
                pltpu.async_copy(dst_h.at[w, s], dst_i.at[s], sd)

            @pl.loop(0, BLOCKS, step=2)
            def _blk(b):
                for s, (gb, ab, sg, se, sd) in enumerate(slots):
                    bb = b + s
                    pltpu.make_async_copy(ht.at[src_v.at[bb]], gb, sg).wait()
                    pltpu.make_async_copy(
                        et.at[pl.ds(ebase + bb * EB, EB)], ab, se).wait()
                    pltpu.make_async_copy(
                        dst_h.at[w, bb], dst_i.at[s], sd).wait()

                    @pl.loop(0, EB)
                    def _row(r):
                        for j in range(CW // 16):
                            sl = pl.ds(j * 16, 16)
                            gb[r, sl] = jnp.maximum(gb[r, sl] + ab[r, sl], 0.0)

                    pltpu.sync_copy(gb, accum.at[dst_i.at[s]], add=True)

                    @pl.when(bb + 2 < BLOCKS)
                    def _issue_next():
                        pltpu.async_copy(ht.at[src_v.at[bb + 2]], gb, sg)
                        pltpu.async_copy(
                            et.at[pl.ds(ebase + (bb + 2) * EB, EB)], ab, se)
                        pltpu.async_copy(dst_h.at[w, bb + 2], dst_i.at[s], sd)

            plsc.subcore_barrier()
            pltpu.sync_copy(accum.at[pl.ds(sid * RPT, RPT)],
                            out_h.at[cid, c, pl.ds(sid * RPT, RPT)])
            plsc.subcore_barrier()

    return k(*hs, *eas, src3, dst3)


_RB = 1000


def _node_proj(x, W, b):
    def body(x_ref, w_ref, b_ref, o0, o1, o2, o3):
        v = jnp.dot(x_ref[...], w_ref[...],
                    preferred_element_type=jnp.float32) + b_ref[...]
        for c, o in enumerate((o0, o1, o2, o3)):
            o[...] = v[:, c * CW:(c + 1) * CW]

    return pl.pallas_call(
        body,
        grid=(N // _RB,),
        in_specs=[
            pl.BlockSpec((_RB, NODE_IN), lambda i: (i, 0)),
            pl.BlockSpec((NODE_IN, H), lambda i: (0, 0)),
            pl.BlockSpec((1, H), lambda i: (0, 0)),
        ],
        out_specs=[pl.BlockSpec((_RB, CW), lambda i: (i, 0))] * CH,
        out_shape=[jax.ShapeDtypeStruct((N, CW), jnp.float32)] * CH,
    )(x, W, b.reshape(1, H))


def _edge_proj_chunk(eap, Wc, bc):
    BE = 2048

    def body(a_ref, w_ref, b_ref, o_ref):
        e = pl.program_id(0)
        v = jnp.dot(a_ref[...], w_ref[...],
                    preferred_element_type=jnp.float32) + b_ref[...]
        row = e * BE + lax.broadcasted_iota(jnp.int32, (BE, CW), 0)
        o_ref[...] = jnp.where(row < E, v, NEG)

    return pl.pallas_call(
        body,
        grid=(EPAD // BE,),
        in_specs=[
            pl.BlockSpec((BE, EDGE_IN), lambda e: (e, 0)),
            pl.BlockSpec((EDGE_IN, CW), lambda e: (0, 0)),
            pl.BlockSpec((1, CW), lambda e: (0, 0)),
        ],
        out_specs=pl.BlockSpec((BE, CW), lambda e: (e, 0)),
        out_shape=jax.ShapeDtypeStruct((EPAD, CW), jnp.float32),
    )(eap, Wc, bc.reshape(1, CW))


def _layer_dense(hs, part, scale, W1, b1, W2, b2, last):
    def body(s_ref, h0, h1, h2, h3, p_ref, w1_ref, b1_ref, w2_ref, b2_ref,
             o0, o1, o2, o3):
        hh = (h0, h1, h2, h3)
        acc = jnp.zeros((_RB, 2 * H), jnp.float32)
        for c in range(CH):
            z = s_ref[...] * hh[c][...] + p_ref[0, c] + p_ref[1, c]
            acc += jnp.dot(z, w1_ref[c * CW:(c + 1) * CW, :],
                           preferred_element_type=jnp.float32)
        z1 = jnp.maximum(acc + b1_ref[...], 0.0)
        h2v = jnp.dot(z1, w2_ref[...], preferred_element_type=jnp.float32)
        h2v = h2v + b2_ref[...]
        if not last:
            h2v = jnp.maximum(h2v, 0.0)
        for c, o in enumerate((o0, o1, o2, o3)):
            o[...] = h2v[:, c * CW:(c + 1) * CW]

    return pl.pallas_call(
        body,
        grid=(N // _RB,),
        in_specs=[
            pl.BlockSpec((1, CW), lambda i: (0, 0)),
            pl.BlockSpec((_RB, CW), lambda i: (i, 0)),
            pl.BlockSpec((_RB, CW), lambda i: (i, 0)),
            pl.BlockSpec((_RB, CW), lambda i: (i, 0)),
            pl.BlockSpec((_RB, CW), lambda i: (i, 0)),
            pl.BlockSpec((NC, CH, _RB, CW), lambda i: (0, 0, i, 0)),
            pl.BlockSpec((H, 2 * H), lambda i: (0, 0)),
            pl.BlockSpec((1, 2 * H), lambda i: (0, 0)),
            pl.BlockSpec((2 * H, H), lambda i: (0, 0)),
            pl.BlockSpec((1, H), lambda i: (0, 0)),
        ],
        out_specs=[pl.BlockSpec((_RB, CW), lambda i: (i, 0))] * CH,
        out_shape=[jax.ShapeDtypeStruct((N, CW), jnp.float32)] * CH,
    )(scale, *hs, part, W1, b1.reshape(1, 2 * H), W2, b2.reshape(1, H))


def _final_mlp(hs, W1, b1, W2p, b2p):
    def body(h0, h1, h2, h3, w1_ref, b1_ref, w2_ref, b2_ref, o_ref):
        hh = (h0, h1, h2, h3)
        acc = jnp.zeros((_RB, H), jnp.float32)
        for c in range(CH):
            acc += jnp.dot(hh[c][...], w1_ref[c * CW:(c + 1) * CW, :],
                           preferred_element_type=jnp.float32)
        z = jnp.maximum(acc + b1_ref[...], 0.0)
        o = jnp.dot(z, w2_ref[...], preferred_element_type=jnp.float32)
        o_ref[...] = jax.nn.sigmoid(o + b2_ref[...])

    return pl.pallas_call(
        body,
        grid=(N // _RB,),
        in_specs=[
            pl.BlockSpec((_RB, CW), lambda i: (i, 0)),
            pl.BlockSpec((_RB, CW), lambda i: (i, 0)),
            pl.BlockSpec((_RB, CW), lambda i: (i, 0)),
            pl.BlockSpec((_RB, CW), lambda i: (i, 0)),
            pl.BlockSpec((H, H), lambda i: (0, 0)),
            pl.BlockSpec((1, H), lambda i: (0, 0)),
            pl.BlockSpec((H, CW), lambda i: (0, 0)),
            pl.BlockSpec((1, CW), lambda i: (0, 0)),
        ],
        out_specs=pl.BlockSpec((_RB, CW), lambda i: (i, 0)),
        out_shape=jax.ShapeDtypeStruct((N, CW), jnp.float32),
    )(*hs, W1, b1.reshape(1, H), W2p, b2p)


def kernel(x, edge_index, edge_attr, W_node, b_node, W_edge, b_edge,
           conv_eps, conv_W1, conv_b1, conv_W2, conv_b2,
           mlp_W1, mlp_b1, mlp_W2, mlp_b2):
    src = edge_index[0]
    dst = edge_index[1]
    pad = EPAD - E
    src3 = jnp.concatenate([src, jnp.zeros((pad,), jnp.int32)]).reshape(
        NW, BLOCKS, EB)
    dst3 = jnp.concatenate([dst, jnp.zeros((pad,), jnp.int32)]).reshape(
        NW, BLOCKS, EB)
    eap = jnp.concatenate(
        [edge_attr, jnp.zeros((pad, EDGE_IN), jnp.float32)], axis=0)

    hs = _node_proj(x, W_node, b_node)
    eas = tuple(
        _edge_proj_chunk(eap, W_edge[:, c * CW:(c + 1) * CW],
                         b_edge[c * CW:(c + 1) * CW])
        for c in range(CH))

    for l in range(L):
        part = _sc_segment_sum(hs, eas, src3, dst3)
        scale = jnp.full((1, CW), 1.0, jnp.float32) + conv_eps[l]
        hs = _layer_dense(hs, part, scale, conv_W1[l], conv_b1[l],
                          conv_W2[l], conv_b2[l], last=(l == L - 1))

    W2p = jnp.pad(mlp_W2, ((0, 0), (0, CW - 1)))
    b2p = jnp.pad(mlp_b2, (0, CW - 1)).reshape(1, CW)
    o = _final_mlp(hs, mlp_W1, mlp_b1, W2p, b2p)
    return o[:, 0]
